# baseline (speedup 1.0000x reference)
import functools
import math

import jax
import jax.numpy as jnp
import numpy as np
from jax import lax
from jax.experimental import pallas as pl
from jax.experimental.pallas import tpu as pltpu
from jax.experimental.pallas import tpu_sc as plsc

_D_MODEL = 1024
_MAX_LEN = 2048
_BATCH = 4
_SEQ = 2048


def _pe_table(max_len, d_model):
    pe = np.zeros((max_len, d_model), dtype=np.float32)
    pos = np.arange(max_len, dtype=np.float32)[:, None]
    div = np.exp(
        np.arange(0, d_model, 2, dtype=np.float32) * (-math.log(10000.0) / d_model)
    )
    pe[:, 0::2] = np.sin(pos * div)
    pe[:, 1::2] = np.cos(pos * div[: d_model // 2])
    return pe


_PE = _pe_table(_MAX_LEN, _D_MODEL)

_NC = 2
_NS = 16
_NW = _NC * _NS
_LANES = 16
_ROWS_W = _SEQ // _NW
_R = 16
_NSUB = _ROWS_W // _R
_ND = _D_MODEL // _LANES
_DC = 16

_mesh = plsc.VectorSubcoreMesh(core_axis_name="c", subcore_axis_name="s")


@functools.partial(
    pl.kernel,
    mesh=_mesh,
    out_type=jax.ShapeDtypeStruct((_BATCH, _SEQ, _D_MODEL), jnp.float32),
    scratch_types=[
        pltpu.VMEM((2, _D_MODEL), jnp.float32),
        pltpu.VMEM((_BATCH, _ROWS_W + _LANES), jnp.int32),
        pltpu.VMEM((_R, _D_MODEL), jnp.float32),
        pltpu.VMEM((_R, _D_MODEL), jnp.float32),
    ],
)
def _sc_embed(x_hbm, tab_hbm, pe_hbm, out_hbm, tb_v, x_v, pe_v, o_v):
    wid = lax.axis_index("s") * _NC + lax.axis_index("c")
    l0 = wid * _ROWS_W
    pltpu.sync_copy(tab_hbm, tb_v)
    pltpu.sync_copy(x_hbm.at[:, pl.ds(l0, _ROWS_W)], x_v.at[:, pl.ds(0, _ROWS_W)])
    def sub_body(i, carry):
        pltpu.sync_copy(pe_hbm.at[pl.ds(l0 + i * _R, _R)], pe_v)

        def batch_body(b, carry2):
            for dc in range(0, _ND, _DC):
                bases = [
                    tb_v[0, pl.ds((dc + j) * _LANES, _LANES)] for j in range(_DC)
                ]
                diffs = [
                    tb_v[1, pl.ds((dc + j) * _LANES, _LANES)] - bases[j]
                    for j in range(_DC)
                ]

                def row_body(r, carry3, dc=dc, bases=bases, diffs=diffs):
                    xv = x_v[b, pl.ds(i * _R + r, _LANES)]
                    w = xv[0].astype(jnp.float32)
                    for j in range(_DC):
                        sl = pl.ds((dc + j) * _LANES, _LANES)
                        o_v[r, sl] = pe_v[r, sl] + bases[j] + w * diffs[j]
                    return carry3

                lax.fori_loop(0, _R, row_body, 0)
            pltpu.sync_copy(o_v, out_hbm.at[b, pl.ds(l0 + i * _R, _R)])
            return carry2

        lax.fori_loop(0, _BATCH, batch_body, 0)
        return carry

    lax.fori_loop(0, _NSUB, sub_body, 0)


def kernel(x, embed_table):
    return _sc_embed(x, embed_table, _PE)
